# baseline (speedup 1.0000x reference)
import functools

import jax
import jax.numpy as jnp
from jax.experimental import pallas as pl
from jax.experimental.pallas import tpu as pltpu

_NCHUNK = 8
_AP = 32


def _mlp_kernel(x_ref, w1_ref, b1_ref, w2_ref, b2_ref, o_ref):
    tb = x_ref.shape[0]
    a = o_ref.shape[-1]
    rs = tb // _NCHUNK
    w1a = jnp.concatenate([w1_ref[...], b1_ref[...]], axis=0)
    w1a = w1a.astype(jnp.bfloat16)
    w2 = w2_ref[...][:, :_AP]
    b2 = b2_ref[0:1, :a]
    for c in range(_NCHUNK):
        xc = x_ref[c * rs:(c + 1) * rs, :]
        ones = jnp.ones((rs, 1), xc.dtype)
        xa = jnp.concatenate([xc, ones], axis=-1).astype(jnp.bfloat16)
        ht = jax.lax.dot_general(
            w1a, xa, (((0,), (1,)), ((), ())),
            preferred_element_type=jnp.float32,
        )
        ht = jnp.maximum(ht, 0.0)
        qt = jax.lax.dot_general(
            w2, ht, (((0,), (0,)), ((), ())),
            preferred_element_type=jnp.float32,
        )
        o_ref[c * rs:(c + 1) * rs, :] = qt[:a, :].T + b2


@functools.partial(jax.jit, static_argnames=("num_actions", "tb"))
def _forward(x, w1p, b1p, w2p, b2p, *, num_actions, tb):
    B, K = x.shape
    H = w1p.shape[1]
    Ap = w2p.shape[1]
    A = num_actions

    grid = (B // tb,)
    cost = pl.CostEstimate(
        flops=2 * B * (K * H + H * _AP),
        transcendentals=0,
        bytes_accessed=4 * (B * K + B * A + K * H + H * _AP + H + _AP),
    )

    return pl.pallas_call(
        _mlp_kernel,
        out_shape=jax.ShapeDtypeStruct((B, A), x.dtype),
        grid=grid,
        in_specs=[
            pl.BlockSpec((tb, K), lambda i: (i, 0)),
            pl.BlockSpec((K, H), lambda i: (0, 0)),
            pl.BlockSpec((1, H), lambda i: (0, 0)),
            pl.BlockSpec((H, Ap), lambda i: (0, 0)),
            pl.BlockSpec((1, Ap), lambda i: (0, 0)),
        ],
        out_specs=pl.BlockSpec((tb, A), lambda i: (i, 0)),
        compiler_params=pltpu.CompilerParams(
            dimension_semantics=("arbitrary",),
            vmem_limit_bytes=96 * 1024 * 1024,
        ),
        cost_estimate=cost,
    )(x, w1p, b1p, w2p, b2p)


def kernel(x, w1p, b1p, w2p, b2p):
    return _forward(x, w1p, b1p, w2p, b2p, num_actions=18, tb=16384)


# speedup vs baseline: 1.0085x; 1.0085x over previous
import functools

import jax
import jax.numpy as jnp
from jax.experimental import pallas as pl
from jax.experimental.pallas import tpu as pltpu

_NCHUNK = 4
_AP = 32


def _mlp_kernel(x_ref, w1_ref, b1_ref, w2_ref, b2_ref, o_ref):
    tb = x_ref.shape[0]
    a = o_ref.shape[-1]
    rs = tb // _NCHUNK
    w1a = jnp.concatenate([w1_ref[...], b1_ref[...]], axis=0)
    w1a = w1a.astype(jnp.bfloat16)
    w2 = w2_ref[...][:, :_AP]
    b2 = b2_ref[0:1, :a]
    for c in range(_NCHUNK):
        xc = x_ref[c * rs:(c + 1) * rs, :]
        ones = jnp.ones((rs, 1), xc.dtype)
        xa = jnp.concatenate([xc, ones], axis=-1).astype(jnp.bfloat16)
        ht = jax.lax.dot_general(
            w1a, xa, (((0,), (1,)), ((), ())),
            preferred_element_type=jnp.float32,
        )
        ht = jnp.maximum(ht, 0.0)
        qt = jax.lax.dot_general(
            w2, ht, (((0,), (0,)), ((), ())),
            preferred_element_type=jnp.float32,
        )
        o_ref[c * rs:(c + 1) * rs, :] = qt[:a, :].T + b2


@functools.partial(jax.jit, static_argnames=("num_actions", "tb"))
def _forward(x, w1p, b1p, w2p, b2p, *, num_actions, tb):
    B, K = x.shape
    H = w1p.shape[1]
    Ap = w2p.shape[1]
    A = num_actions

    grid = (B // tb,)
    cost = pl.CostEstimate(
        flops=2 * B * (K * H + H * _AP),
        transcendentals=0,
        bytes_accessed=4 * (B * K + B * A + K * H + H * _AP + H + _AP),
    )

    return pl.pallas_call(
        _mlp_kernel,
        out_shape=jax.ShapeDtypeStruct((B, A), x.dtype),
        grid=grid,
        in_specs=[
            pl.BlockSpec((tb, K), lambda i: (i, 0)),
            pl.BlockSpec((K, H), lambda i: (0, 0)),
            pl.BlockSpec((1, H), lambda i: (0, 0)),
            pl.BlockSpec((H, Ap), lambda i: (0, 0)),
            pl.BlockSpec((1, Ap), lambda i: (0, 0)),
        ],
        out_specs=pl.BlockSpec((tb, A), lambda i: (i, 0)),
        compiler_params=pltpu.CompilerParams(
            dimension_semantics=("arbitrary",),
            vmem_limit_bytes=96 * 1024 * 1024,
        ),
        cost_estimate=cost,
    )(x, w1p, b1p, w2p, b2p)


def kernel(x, w1p, b1p, w2p, b2p):
    return _forward(x, w1p, b1p, w2p, b2p, num_actions=18, tb=16384)
